# Initial kernel scaffold; baseline (speedup 1.0000x reference)
import jax
import jax.numpy as jnp
from jax.experimental import pallas as pl


def kernel(phi_indices, phi_values, phi_inverse_indices, phi_inverse_values, features, weight_matrix, diagonal_weight_filter):
    raise NotImplementedError("write your pallas kernel here")


# speedup vs baseline: 5.2061x; 5.2061x over previous
import functools

import jax
import jax.numpy as jnp
from jax import lax
from jax.experimental import pallas as pl
from jax.experimental.pallas import tpu as pltpu
from jax.experimental.pallas import tpu_sc as plsc

N = 10000
NNZ = 320000
D = 128

NC = 2
NS = 16
NW = NC * NS
EPT = NNZ // NW
CH = 128
NCHUNK = 80
EPTP = NCHUNK * CH
RPT = 624
RZ = 16


def _zero_fill(zbuf, nrows):
    z16 = jnp.zeros((16,), jnp.float32)
    for r in range(nrows):
        for q in range(D // 16):
            zbuf[r, pl.ds(q * 16, 16)] = z16


_GDN = lax.GatherDimensionNumbers(
    offset_dims=(), collapsed_slice_dims=(0,), start_index_map=(0,))


def _splat(vec, e):
    idx = jnp.full((16, 1), e, jnp.int32)
    return lax.gather(vec, idx, _GDN, slice_sizes=(1,),
                      mode=lax.GatherScatterMode.PROMISE_IN_BOUNDS)


def _scale_rows(gbuf, vals_v, j):
    for g in range(CH // 16):
        vv = vals_v[pl.ds(j * CH + g * 16, 16)]
        for e in range(16):
            sp = _splat(vv, e)
            row = g * 16 + e
            for q in range(D // 16):
                gbuf[row, pl.ds(q * 16, 16)] = (
                    gbuf[row, pl.ds(q * 16, 16)] * sp)


_sc_mesh = plsc.VectorSubcoreMesh(core_axis_name="c", subcore_axis_name="s")


@functools.partial(
    pl.kernel,
    out_type=jax.ShapeDtypeStruct((NC, N, D), jnp.float32),
    mesh=_sc_mesh,
    scratch_types=[
        pltpu.VMEM((NCHUNK, CH), jnp.int32),
        pltpu.VMEM((NCHUNK, CH), jnp.int32),
        pltpu.VMEM((EPTP,), jnp.float32),
        pltpu.VMEM((CH, D), jnp.float32),
        pltpu.VMEM((RZ, D), jnp.float32),
        pltpu.VMEM_SHARED((N, D), jnp.float32),
        pltpu.SemaphoreType.DMA,
    ],
)
def _spmm_sc(cols_hbm, rows_hbm, vals_hbm, x_hbm, out_hbm,
             cols_v, rows_v, vals_v, gbuf0, zbuf, acc, sem0):
    c = lax.axis_index("c")
    s = lax.axis_index("s")
    wid = c * NS + s

    pltpu.sync_copy(cols_hbm.at[wid], cols_v)
    pltpu.sync_copy(rows_hbm.at[wid], rows_v)
    pltpu.sync_copy(vals_hbm.at[wid], vals_v)

    base = s * RPT
    _zero_fill(zbuf, RZ)
    def zc(k, carry):
        pltpu.sync_copy(zbuf, acc.at[pl.ds(base + k * RZ, RZ)])
        return carry
    lax.fori_loop(0, RPT // RZ, zc, 0)
    @pl.when(s == NS - 1)
    def _():
        pltpu.sync_copy(zbuf, acc.at[pl.ds(NS * RPT, RZ)])
    plsc.subcore_barrier()

    def body(j, carry):
        pltpu.async_copy(x_hbm.at[cols_v.at[j]], gbuf0, sem0).wait()
        _scale_rows(gbuf0, vals_v, j)
        pltpu.sync_copy(gbuf0, acc.at[rows_v.at[j]], add=True)
        return carry
    lax.fori_loop(0, NCHUNK, body, 0)

    plsc.subcore_barrier()
    pltpu.sync_copy(acc.at[pl.ds(base, RPT)],
                    out_hbm.at[c, pl.ds(base, RPT)])
    @pl.when(s == NS - 1)
    def _():
        pltpu.sync_copy(acc.at[pl.ds(NS * RPT, RZ)],
                        out_hbm.at[c, pl.ds(NS * RPT, RZ)])


def _matmul_body(x_ref, w_ref, o_ref):
    o_ref[...] = jnp.dot(x_ref[...], w_ref[...],
                         preferred_element_type=jnp.float32)


def _combine_theta_body(p_ref, t_ref, o_ref):
    o_ref[...] = (p_ref[0] + p_ref[1]) * t_ref[...]


def _combine_body(p_ref, o_ref):
    o_ref[...] = p_ref[0] + p_ref[1]


_BM = 1000


def _matmul(x, w):
    return pl.pallas_call(
        _matmul_body,
        grid=(N // _BM,),
        in_specs=[
            pl.BlockSpec((_BM, D), lambda i: (i, 0)),
            pl.BlockSpec((D, D), lambda i: (0, 0)),
        ],
        out_specs=pl.BlockSpec((_BM, D), lambda i: (i, 0)),
        out_shape=jax.ShapeDtypeStruct((N, D), jnp.float32),
    )(x, w)


def _combine_theta(p, theta):
    return pl.pallas_call(
        _combine_theta_body,
        grid=(N // _BM,),
        in_specs=[
            pl.BlockSpec((NC, _BM, D), lambda i: (0, i, 0)),
            pl.BlockSpec((_BM, 1), lambda i: (i, 0)),
        ],
        out_specs=pl.BlockSpec((_BM, D), lambda i: (i, 0)),
        out_shape=jax.ShapeDtypeStruct((N, D), jnp.float32),
    )(p, theta)


def _combine(p):
    return pl.pallas_call(
        _combine_body,
        grid=(N // _BM,),
        in_specs=[pl.BlockSpec((NC, _BM, D), lambda i: (0, i, 0))],
        out_specs=pl.BlockSpec((_BM, D), lambda i: (i, 0)),
        out_shape=jax.ShapeDtypeStruct((N, D), jnp.float32),
    )(p)


def _prep_edges(indices, values):
    rows = indices[0].reshape(NW, EPT)
    cols = indices[1].reshape(NW, EPT)
    vals = values.reshape(NW, EPT)
    pad = EPTP - EPT
    rows = jnp.pad(rows, ((0, 0), (0, pad)))
    cols = jnp.pad(cols, ((0, 0), (0, pad)))
    vals = jnp.pad(vals, ((0, 0), (0, pad)))
    return (cols.reshape(NW, NCHUNK, CH), rows.reshape(NW, NCHUNK, CH),
            vals)


@jax.jit
def kernel(phi_indices, phi_values, phi_inverse_indices, phi_inverse_values,
           features, weight_matrix, diagonal_weight_filter):
    x = features[:, 0, :]
    filtered = _matmul(x, weight_matrix)

    inv_cols, inv_rows, inv_vals = _prep_edges(
        phi_inverse_indices, phi_inverse_values)
    p1 = _spmm_sc(inv_cols, inv_rows, inv_vals, filtered)

    tmp_scaled = _combine_theta(p1, diagonal_weight_filter)

    phi_cols, phi_rows, phi_vals = _prep_edges(phi_indices, phi_values)
    p2 = _spmm_sc(phi_cols, phi_rows, phi_vals, tmp_scaled)

    out = _combine(p2)
    return out[:, None, :]


# speedup vs baseline: 5.4772x; 1.0521x over previous
import functools

import jax
import jax.numpy as jnp
from jax import lax
from jax.experimental import pallas as pl
from jax.experimental.pallas import tpu as pltpu
from jax.experimental.pallas import tpu_sc as plsc

N = 10000
NNZ = 320000
D = 128

NC = 2
NS = 16
NW = NC * NS
EPT = NNZ // NW
CH = 128
NCHUNK = 80
EPTP = NCHUNK * CH
RPT = 624
RZ = 16


def _zero_fill(zbuf, nrows):
    z16 = jnp.zeros((16,), jnp.float32)
    for r in range(nrows):
        for q in range(D // 16):
            zbuf[r, pl.ds(q * 16, 16)] = z16


_GDN = lax.GatherDimensionNumbers(
    offset_dims=(), collapsed_slice_dims=(0,), start_index_map=(0,))


def _splat(vec, e):
    idx = jnp.full((16, 1), e, jnp.int32)
    return lax.gather(vec, idx, _GDN, slice_sizes=(1,),
                      mode=lax.GatherScatterMode.PROMISE_IN_BOUNDS)


def _scale_rows(gbuf, vals_v, j):
    def grp(g, carry):
        vv = vals_v[pl.ds(j * CH + g * 16, 16)]
        row = g * 16
        for e in range(16):
            sp = _splat(vv, e)
            for q in range(D // 16):
                gbuf[row + e, pl.ds(q * 16, 16)] = (
                    gbuf[row + e, pl.ds(q * 16, 16)] * sp)
        return carry
    lax.fori_loop(0, CH // 16, grp, 0)


_sc_mesh = plsc.VectorSubcoreMesh(core_axis_name="c", subcore_axis_name="s")


@functools.partial(
    pl.kernel,
    out_type=jax.ShapeDtypeStruct((NC, N, D), jnp.float32),
    mesh=_sc_mesh,
    scratch_types=[
        pltpu.VMEM((EPTP,), jnp.float32),
        pltpu.VMEM((CH,), jnp.int32),
        pltpu.VMEM((CH,), jnp.int32),
        pltpu.VMEM((CH,), jnp.int32),
        pltpu.VMEM((CH,), jnp.int32),
        pltpu.VMEM((CH, D), jnp.float32),
        pltpu.VMEM((CH, D), jnp.float32),
        pltpu.VMEM((RZ, D), jnp.float32),
        pltpu.VMEM_SHARED((N, D), jnp.float32),
        pltpu.SemaphoreType.DMA,
        pltpu.SemaphoreType.DMA,
        pltpu.SemaphoreType.DMA,
        pltpu.SemaphoreType.DMA,
        pltpu.SemaphoreType.DMA,
        pltpu.SemaphoreType.DMA,
    ],
)
def _spmm_sc(cols_hbm, rows_hbm, vals_hbm, x_hbm, out_hbm,
             vals_v, cbuf0, cbuf1, rbuf0, rbuf1, gbuf0, gbuf1, zbuf, acc,
             csem0, csem1, rsem0, rsem1, gsem0, gsem1):
    c = lax.axis_index("c")
    s = lax.axis_index("s")
    wid = c * NS + s

    pltpu.sync_copy(vals_hbm.at[wid], vals_v)

    base = s * RPT
    _zero_fill(zbuf, RZ)
    def zc(k, carry):
        pltpu.sync_copy(zbuf, acc.at[pl.ds(base + k * RZ, RZ)])
        return carry
    lax.fori_loop(0, RPT // RZ, zc, 0)
    @pl.when(s == NS - 1)
    def _():
        pltpu.sync_copy(zbuf, acc.at[pl.ds(NS * RPT, RZ)])
    plsc.subcore_barrier()

    def fire_cols(j, cbuf, csem):
        pltpu.async_copy(cols_hbm.at[wid, j], cbuf, csem)

    def fire_rows(j, rbuf, rsem):
        pltpu.async_copy(rows_hbm.at[wid, j], rbuf, rsem)

    def wait(hbm_src, buf, sem):
        pltpu.make_async_copy(hbm_src, buf, sem).wait()

    def fire_gather(cbuf, gbuf, gsem):
        pltpu.async_copy(x_hbm.at[cbuf], gbuf, gsem)

    fire_cols(0, cbuf0, csem0)
    fire_rows(0, rbuf0, rsem0)
    fire_cols(1, cbuf1, csem1)
    fire_rows(1, rbuf1, rsem1)
    wait(cols_hbm.at[wid, 0], cbuf0, csem0)
    fire_gather(cbuf0, gbuf0, gsem0)

    def half(j, cbuf, csem, rbuf, rsem, gbuf, gsem, jn):
        wait(x_hbm.at[cbuf], gbuf, gsem)
        fire_cols(jn, cbuf, csem)
        _scale_rows(gbuf, vals_v, j)
        wait(rows_hbm.at[wid, 0], rbuf, rsem)
        pltpu.sync_copy(gbuf, acc.at[rbuf], add=True)
        fire_rows(jn, rbuf, rsem)

    def body(p, carry):
        j0 = 2 * p
        wait(cols_hbm.at[wid, 0], cbuf1, csem1)
        fire_gather(cbuf1, gbuf1, gsem1)
        half(j0, cbuf0, csem0, rbuf0, rsem0, gbuf0, gsem0, j0 + 2)
        half(j0 + 1, cbuf1, csem1, rbuf1, rsem1, gbuf1, gsem1, j0 + 3)
        wait(cols_hbm.at[wid, 0], cbuf0, csem0)
        fire_gather(cbuf0, gbuf0, gsem0)
        return carry
    lax.fori_loop(0, NCHUNK // 2 - 1, body, 0)

    wait(cols_hbm.at[wid, 0], cbuf1, csem1)
    fire_gather(cbuf1, gbuf1, gsem1)
    wait(x_hbm.at[cbuf0], gbuf0, gsem0)
    _scale_rows(gbuf0, vals_v, NCHUNK - 2)
    wait(rows_hbm.at[wid, 0], rbuf0, rsem0)
    pltpu.sync_copy(gbuf0, acc.at[rbuf0], add=True)
    wait(x_hbm.at[cbuf1], gbuf1, gsem1)
    _scale_rows(gbuf1, vals_v, NCHUNK - 1)
    wait(rows_hbm.at[wid, 0], rbuf1, rsem1)
    pltpu.sync_copy(gbuf1, acc.at[rbuf1], add=True)

    plsc.subcore_barrier()
    pltpu.sync_copy(acc.at[pl.ds(base, RPT)],
                    out_hbm.at[c, pl.ds(base, RPT)])
    @pl.when(s == NS - 1)
    def _():
        pltpu.sync_copy(acc.at[pl.ds(NS * RPT, RZ)],
                        out_hbm.at[c, pl.ds(NS * RPT, RZ)])


def _matmul_body(x_ref, w_ref, o_ref):
    o_ref[...] = jnp.dot(x_ref[...], w_ref[...],
                         preferred_element_type=jnp.float32)


def _combine_theta_body(p_ref, t_ref, o_ref):
    o_ref[...] = (p_ref[0] + p_ref[1]) * t_ref[...]


def _combine_body(p_ref, o_ref):
    o_ref[...] = p_ref[0] + p_ref[1]


_BM = 1000


def _matmul(x, w):
    return pl.pallas_call(
        _matmul_body,
        grid=(N // _BM,),
        in_specs=[
            pl.BlockSpec((_BM, D), lambda i: (i, 0)),
            pl.BlockSpec((D, D), lambda i: (0, 0)),
        ],
        out_specs=pl.BlockSpec((_BM, D), lambda i: (i, 0)),
        out_shape=jax.ShapeDtypeStruct((N, D), jnp.float32),
    )(x, w)


def _combine_theta(p, theta):
    return pl.pallas_call(
        _combine_theta_body,
        grid=(N // _BM,),
        in_specs=[
            pl.BlockSpec((NC, _BM, D), lambda i: (0, i, 0)),
            pl.BlockSpec((_BM, 1), lambda i: (i, 0)),
        ],
        out_specs=pl.BlockSpec((_BM, D), lambda i: (i, 0)),
        out_shape=jax.ShapeDtypeStruct((N, D), jnp.float32),
    )(p, theta)


def _combine(p):
    return pl.pallas_call(
        _combine_body,
        grid=(N // _BM,),
        in_specs=[pl.BlockSpec((NC, _BM, D), lambda i: (0, i, 0))],
        out_specs=pl.BlockSpec((_BM, D), lambda i: (i, 0)),
        out_shape=jax.ShapeDtypeStruct((N, D), jnp.float32),
    )(p)


def _prep_edges(indices, values):
    rows = indices[0].reshape(NW, EPT)
    cols = indices[1].reshape(NW, EPT)
    vals = values.reshape(NW, EPT)
    pad = EPTP - EPT
    rows = jnp.pad(rows, ((0, 0), (0, pad)))
    cols = jnp.pad(cols, ((0, 0), (0, pad)))
    vals = jnp.pad(vals, ((0, 0), (0, pad)))
    return (cols.reshape(NW, NCHUNK, CH), rows.reshape(NW, NCHUNK, CH),
            vals)


@jax.jit
def kernel(phi_indices, phi_values, phi_inverse_indices, phi_inverse_values,
           features, weight_matrix, diagonal_weight_filter):
    x = features[:, 0, :]
    filtered = _matmul(x, weight_matrix)

    inv_cols, inv_rows, inv_vals = _prep_edges(
        phi_inverse_indices, phi_inverse_values)
    p1 = _spmm_sc(inv_cols, inv_rows, inv_vals, filtered)

    tmp_scaled = _combine_theta(p1, diagonal_weight_filter)

    phi_cols, phi_rows, phi_vals = _prep_edges(phi_indices, phi_values)
    p2 = _spmm_sc(phi_cols, phi_rows, phi_vals, tmp_scaled)

    out = _combine(p2)
    return out[:, None, :]


# speedup vs baseline: 12.4228x; 2.2681x over previous
import functools

import jax
import jax.numpy as jnp
from jax import lax
from jax.experimental import pallas as pl
from jax.experimental.pallas import tpu as pltpu
from jax.experimental.pallas import tpu_sc as plsc

N = 10000
NNZ = 320000
D = 128

NC = 2
NS = 16
NW = NC * NS
EPT = NNZ // NW
CH = 128
NCHUNK = 80
EPTP = NCHUNK * CH
RPT = 624
RZ = 16


def _zero_fill(zbuf, nrows):
    z16 = jnp.zeros((16,), jnp.float32)
    for r in range(nrows):
        for q in range(D // 16):
            zbuf[r, pl.ds(q * 16, 16)] = z16


_GDN = lax.GatherDimensionNumbers(
    offset_dims=(), collapsed_slice_dims=(0,), start_index_map=(0,))


def _splat(vec, e):
    idx = jnp.full((16, 1), e, jnp.int32)
    return lax.gather(vec, idx, _GDN, slice_sizes=(1,),
                      mode=lax.GatherScatterMode.PROMISE_IN_BOUNDS)


def _scale_rows(gbuf, vbuf):
    def grp(g, carry):
        vv = vbuf[pl.ds(g * 16, 16)]
        row = g * 16
        for e in range(16):
            sp = _splat(vv, e)
            for q in range(D // 16):
                gbuf[row + e, pl.ds(q * 16, 16)] = (
                    gbuf[row + e, pl.ds(q * 16, 16)] * sp)
        return carry
    lax.fori_loop(0, CH // 16, grp, 0)


_sc_mesh = plsc.VectorSubcoreMesh(core_axis_name="c", subcore_axis_name="s")


@functools.partial(
    pl.kernel,
    out_type=jax.ShapeDtypeStruct((NW, NCHUNK, CH, D), jnp.float32),
    mesh=_sc_mesh,
    scratch_types=[
        pltpu.VMEM((CH,), jnp.int32),
        pltpu.VMEM((CH,), jnp.int32),
        pltpu.VMEM((CH,), jnp.int32),
        pltpu.VMEM((CH,), jnp.float32),
        pltpu.VMEM((CH,), jnp.float32),
        pltpu.VMEM((CH,), jnp.float32),
        pltpu.VMEM((CH, D), jnp.float32),
        pltpu.VMEM((CH, D), jnp.float32),
        pltpu.VMEM((CH, D), jnp.float32),
        pltpu.VMEM_SHARED((N, D), jnp.float32),
        pltpu.SemaphoreType.DMA,
        pltpu.SemaphoreType.DMA,
        pltpu.SemaphoreType.DMA,
        pltpu.SemaphoreType.DMA,
        pltpu.SemaphoreType.DMA,
        pltpu.SemaphoreType.DMA,
        pltpu.SemaphoreType.DMA,
        pltpu.SemaphoreType.DMA,
        pltpu.SemaphoreType.DMA,
        pltpu.SemaphoreType.DMA,
        pltpu.SemaphoreType.DMA,
        pltpu.SemaphoreType.DMA,
    ],
)
def _gather_scale_sc(cols_hbm, vals_hbm, x_hbm, ebuf_hbm,
                     cbuf0, cbuf1, cbuf2, vbuf0, vbuf1, vbuf2,
                     gbuf0, gbuf1, gbuf2, xs,
                     csem0, csem1, csem2, vsem0, vsem1, vsem2,
                     gsem0, gsem1, gsem2, wsem0, wsem1, wsem2):
    c = lax.axis_index("c")
    s = lax.axis_index("s")
    wid = c * NS + s
    base = s * RPT

    cbuf = (cbuf0, cbuf1, cbuf2)
    vbuf = (vbuf0, vbuf1, vbuf2)
    gbuf = (gbuf0, gbuf1, gbuf2)
    csem = (csem0, csem1, csem2)
    vsem = (vsem0, vsem1, vsem2)
    gsem = (gsem0, gsem1, gsem2)
    wsem = (wsem0, wsem1, wsem2)

    pltpu.sync_copy(x_hbm.at[pl.ds(base, RPT)], xs.at[pl.ds(base, RPT)])
    @pl.when(s == NS - 1)
    def _():
        pltpu.sync_copy(x_hbm.at[pl.ds(NS * RPT, RZ)],
                        xs.at[pl.ds(NS * RPT, RZ)])
    plsc.subcore_barrier()

    def fire_cols(j, k):
        pltpu.async_copy(cols_hbm.at[wid, j], cbuf[k], csem[k])

    def fire_vals(j, k):
        pltpu.async_copy(vals_hbm.at[wid, j], vbuf[k], vsem[k])

    def fire_gather(k):
        pltpu.async_copy(xs.at[cbuf[k]], gbuf[k], gsem[k])

    def fire_write(j, k):
        pltpu.async_copy(gbuf[k], ebuf_hbm.at[wid, j], wsem[k])

    def wait(src, dst, sem):
        pltpu.make_async_copy(src, dst, sem).wait()

    for k in range(3):
        fire_cols(k, k)
        fire_vals(k, k)
    wait(cols_hbm.at[wid, 0], cbuf[0], csem[0])
    fire_gather(0)
    wait(cols_hbm.at[wid, 0], cbuf[1], csem[1])
    fire_gather(1)

    def third(j, k, first, f_cv, f_g):
        kn = (k + 2) % 3
        wait(xs.at[cbuf[k]], gbuf[k], gsem[k])
        if f_cv:
            fire_cols(j + 3, k)
        wait(vals_hbm.at[wid, 0], vbuf[k], vsem[k])
        _scale_rows(gbuf[k], vbuf[k])
        if f_cv:
            fire_vals(j + 3, k)
        if not first:
            wait(gbuf[kn], ebuf_hbm.at[wid, 0], wsem[kn])
        fire_write(j, k)
        if f_g:
            wait(cols_hbm.at[wid, 0], cbuf[kn], csem[kn])
            fire_gather(kn)

    third(0, 0, True, True, True)

    def body_dyn(p, carry):
        j0 = 3 * p
        for (q, k) in ((1, 1), (2, 2), (3, 0)):
            j = j0 + q
            kn = (k + 2) % 3
            wait(xs.at[cbuf[k]], gbuf[k], gsem[k])
            fire_cols(j + 3, k)
            wait(vals_hbm.at[wid, 0], vbuf[k], vsem[k])
            _scale_rows(gbuf[k], vbuf[k])
            fire_vals(j + 3, k)
            wait(gbuf[kn], ebuf_hbm.at[wid, 0], wsem[kn])
            fire_write(j, k)
            wait(cols_hbm.at[wid, 0], cbuf[kn], csem[kn])
            fire_gather(kn)
        return carry
    lax.fori_loop(0, (NCHUNK - 5) // 3, body_dyn, 0)

    third(NCHUNK - 4, 1, False, True, True)
    third(NCHUNK - 3, 2, False, False, True)
    third(NCHUNK - 2, 0, False, False, False)
    third(NCHUNK - 1, 1, False, False, False)
    wait(gbuf[1], ebuf_hbm.at[wid, 0], wsem[1])


@functools.partial(
    pl.kernel,
    out_type=jax.ShapeDtypeStruct((NC, N, D), jnp.float32),
    mesh=_sc_mesh,
    scratch_types=[
        pltpu.VMEM((CH,), jnp.int32),
        pltpu.VMEM((CH,), jnp.int32),
        pltpu.VMEM((CH,), jnp.int32),
        pltpu.VMEM((CH, D), jnp.float32),
        pltpu.VMEM((CH, D), jnp.float32),
        pltpu.VMEM((CH, D), jnp.float32),
        pltpu.VMEM_SHARED((N, D), jnp.float32),
        pltpu.SemaphoreType.DMA,
        pltpu.SemaphoreType.DMA,
        pltpu.SemaphoreType.DMA,
        pltpu.SemaphoreType.DMA,
        pltpu.SemaphoreType.DMA,
        pltpu.SemaphoreType.DMA,
    ],
)
def _scatter_sc(rows_hbm, ebuf_hbm, out_hbm,
                rbuf0, rbuf1, rbuf2, gbuf0, gbuf1, gbuf2, acc,
                rsem0, rsem1, rsem2, dsem0, dsem1, dsem2):
    c = lax.axis_index("c")
    s = lax.axis_index("s")
    wid = c * NS + s
    base = s * RPT

    rbuf = (rbuf0, rbuf1, rbuf2)
    gbuf = (gbuf0, gbuf1, gbuf2)
    rsem = (rsem0, rsem1, rsem2)
    dsem = (dsem0, dsem1, dsem2)

    _zero_fill(gbuf0, RZ)
    zsrc = gbuf0.at[pl.ds(0, RZ)]
    def zc(k, carry):
        pltpu.sync_copy(zsrc, acc.at[pl.ds(base + k * RZ, RZ)])
        return carry
    lax.fori_loop(0, RPT // RZ, zc, 0)
    @pl.when(s == NS - 1)
    def _():
        pltpu.sync_copy(zsrc, acc.at[pl.ds(NS * RPT, RZ)])
    plsc.subcore_barrier()

    def fire_rows(j, k):
        pltpu.async_copy(rows_hbm.at[wid, j], rbuf[k], rsem[k])

    def fire_read(j, k):
        pltpu.async_copy(ebuf_hbm.at[wid, j], gbuf[k], dsem[k])

    def wait(src, dst, sem):
        pltpu.make_async_copy(src, dst, sem).wait()

    for k in range(3):
        fire_rows(k, k)
        fire_read(k, k)

    def third(j, k, guard):
        wait(ebuf_hbm.at[wid, 0], gbuf[k], dsem[k])
        wait(rows_hbm.at[wid, 0], rbuf[k], rsem[k])
        pltpu.sync_copy(gbuf[k], acc.at[rbuf[k]], add=True)
        if guard:
            @pl.when(j + 3 < NCHUNK)
            def _():
                fire_rows(j + 3, k)
                fire_read(j + 3, k)

    def body(p, carry):
        j0 = 3 * p
        third(j0, 0, True)
        third(j0 + 1, 1, True)
        third(j0 + 2, 2, True)
        return carry
    lax.fori_loop(0, NCHUNK // 3, body, 0)

    third(NCHUNK - 2, 0, False)
    third(NCHUNK - 1, 1, False)

    plsc.subcore_barrier()
    pltpu.sync_copy(acc.at[pl.ds(base, RPT)],
                    out_hbm.at[c, pl.ds(base, RPT)])
    @pl.when(s == NS - 1)
    def _():
        pltpu.sync_copy(acc.at[pl.ds(NS * RPT, RZ)],
                        out_hbm.at[c, pl.ds(NS * RPT, RZ)])


def _matmul_body(x_ref, w_ref, o_ref):
    o_ref[...] = jnp.dot(x_ref[...], w_ref[...],
                         preferred_element_type=jnp.float32)


def _combine_theta_body(p_ref, t_ref, o_ref):
    o_ref[...] = (p_ref[0] + p_ref[1]) * t_ref[...]


def _combine_body(p_ref, o_ref):
    o_ref[...] = p_ref[0] + p_ref[1]


_BM = 1000


def _matmul(x, w):
    return pl.pallas_call(
        _matmul_body,
        grid=(N // _BM,),
        in_specs=[
            pl.BlockSpec((_BM, D), lambda i: (i, 0)),
            pl.BlockSpec((D, D), lambda i: (0, 0)),
        ],
        out_specs=pl.BlockSpec((_BM, D), lambda i: (i, 0)),
        out_shape=jax.ShapeDtypeStruct((N, D), jnp.float32),
    )(x, w)


def _combine_theta(p, theta):
    return pl.pallas_call(
        _combine_theta_body,
        grid=(N // _BM,),
        in_specs=[
            pl.BlockSpec((NC, _BM, D), lambda i: (0, i, 0)),
            pl.BlockSpec((_BM, 1), lambda i: (i, 0)),
        ],
        out_specs=pl.BlockSpec((_BM, D), lambda i: (i, 0)),
        out_shape=jax.ShapeDtypeStruct((N, D), jnp.float32),
    )(p, theta)


def _combine(p):
    return pl.pallas_call(
        _combine_body,
        grid=(N // _BM,),
        in_specs=[pl.BlockSpec((NC, _BM, D), lambda i: (0, i, 0))],
        out_specs=pl.BlockSpec((_BM, D), lambda i: (i, 0)),
        out_shape=jax.ShapeDtypeStruct((N, D), jnp.float32),
    )(p)


def _prep_edges(indices, values):
    rows = indices[0].reshape(NW, EPT)
    cols = indices[1].reshape(NW, EPT)
    vals = values.reshape(NW, EPT)
    pad = EPTP - EPT
    rows = jnp.pad(rows, ((0, 0), (0, pad)))
    cols = jnp.pad(cols, ((0, 0), (0, pad)))
    vals = jnp.pad(vals, ((0, 0), (0, pad)))
    return (cols.reshape(NW, NCHUNK, CH), rows.reshape(NW, NCHUNK, CH),
            vals.reshape(NW, NCHUNK, CH))


def _spmm(cols, rows, vals, x):
    ebuf = _gather_scale_sc(cols, vals, x)
    return _scatter_sc(rows, ebuf)


@jax.jit
def kernel(phi_indices, phi_values, phi_inverse_indices, phi_inverse_values,
           features, weight_matrix, diagonal_weight_filter):
    x = features[:, 0, :]
    filtered = _matmul(x, weight_matrix)

    inv_cols, inv_rows, inv_vals = _prep_edges(
        phi_inverse_indices, phi_inverse_values)
    p1 = _spmm(inv_cols, inv_rows, inv_vals, filtered)

    tmp_scaled = _combine_theta(p1, diagonal_weight_filter)

    phi_cols, phi_rows, phi_vals = _prep_edges(phi_indices, phi_values)
    p2 = _spmm(phi_cols, phi_rows, phi_vals, tmp_scaled)

    out = _combine(p2)
    return out[:, None, :]
